# baseline (speedup 1.0000x reference)
import functools

import jax
import jax.numpy as jnp
from jax import lax
from jax.experimental import pallas as pl
from jax.experimental.pallas import tpu as pltpu
from jax.experimental.pallas import tpu_sc as plsc

E = 8
K = 2
NEXP = E + 1
B = 128
BT = 256


def _router_kernel(x_ref, wg_ref, bg_ref, out_ref):
    logits = jnp.dot(x_ref[...], wg_ref[...],
                     preferred_element_type=jnp.float32) + bg_ref[0, :]
    lane = jax.lax.broadcasted_iota(jnp.int32, logits.shape, 1)
    big = jnp.int32(10**9)
    m1 = jnp.max(logits, axis=1, keepdims=True)
    i1 = jnp.min(jnp.where(logits >= m1, lane, big), axis=1, keepdims=True)
    l2 = jnp.where(lane == i1, -jnp.inf, logits)
    m2 = jnp.max(l2, axis=1, keepdims=True)
    i2 = jnp.min(jnp.where(l2 >= m2, lane, big), axis=1, keepdims=True)
    w1 = jax.nn.sigmoid(m1 - m2)
    w2 = jax.nn.sigmoid(m2 - m1)
    out = (jnp.where(lane == 0, i1.astype(jnp.float32), 0.0)
           + jnp.where(lane == 1, i2.astype(jnp.float32), 0.0)
           + jnp.where(lane == 2, w1, 0.0)
           + jnp.where(lane == 3, w2, 0.0))
    out_ref[...] = out[:, :8]


def _route(x, Wg, bg):
    T, H = x.shape
    Wgp = jnp.zeros((H, 128), jnp.float32).at[:, :E].set(Wg)
    bgp = jnp.full((1, 128), -1e30, jnp.float32).at[0, :E].set(bg)
    return pl.pallas_call(
        _router_kernel,
        grid=(T // BT,),
        in_specs=[
            pl.BlockSpec((BT, H), lambda i: (i, 0)),
            pl.BlockSpec((H, 128), lambda i: (0, 0)),
            pl.BlockSpec((1, 128), lambda i: (0, 0)),
        ],
        out_specs=pl.BlockSpec((BT, 8), lambda i: (i, 0)),
        out_shape=jax.ShapeDtypeStruct((T, 8), jnp.float32),
    )(x, Wgp, bgp)


def _mlp1_kernel(bexp_ref, acts_ref, xs_ref, w1g_ref, w1l_ref, b1g_ref,
                 b1l_ref, h_ref):
    e = bexp_ref[pl.program_id(0)]
    xb = xs_ref[...].astype(jnp.bfloat16)
    g = jnp.dot(xb, w1g_ref[0], preferred_element_type=jnp.float32)
    g = g + b1g_ref[0, 0, :]
    l = jnp.dot(xb, w1l_ref[0], preferred_element_type=jnp.float32)
    l = l + b1l_ref[0, 0, :]
    alpha = acts_ref[e, 0]
    gsc = acts_ref[e, 1]
    ush = acts_ref[e, 2]
    gc = jnp.log1p(jnp.exp(jnp.full(g.shape, acts_ref[e, 3], jnp.float32)))
    uc = jnp.log1p(jnp.exp(jnp.full(g.shape, acts_ref[e, 4], jnp.float32)))
    xg = jnp.clip(g, -gc, gc)
    xl = jnp.clip(l, -uc, uc)
    og = xg * jax.nn.sigmoid(xg * alpha) * gsc
    h_ref[...] = (og * (xl + ush)).astype(jnp.bfloat16)


def _mlp2_kernel(bexp_ref, h_ref, w2_ref, b2_ref, ws_ref, y_ref):
    y = jnp.dot(h_ref[...], w2_ref[0], preferred_element_type=jnp.float32)
    y = y + b2_ref[0, 0, :]
    y_ref[...] = y * ws_ref[...][:, :1]


def _grouped_mlp(xs, bexp, acts, W1g, W1l, b1g, b1l, W2, b2, ws8):
    Mtot, H = xs.shape
    I = W1g.shape[2]
    NB = Mtot // B
    h = pl.pallas_call(
        _mlp1_kernel,
        grid_spec=pltpu.PrefetchScalarGridSpec(
            num_scalar_prefetch=2,
            grid=(NB,),
            in_specs=[
                pl.BlockSpec((B, H), lambda i, be, ac: (i, 0)),
                pl.BlockSpec((1, H, I), lambda i, be, ac: (be[i], 0, 0)),
                pl.BlockSpec((1, H, I), lambda i, be, ac: (be[i], 0, 0)),
                pl.BlockSpec((1, 1, I), lambda i, be, ac: (be[i], 0, 0)),
                pl.BlockSpec((1, 1, I), lambda i, be, ac: (be[i], 0, 0)),
            ],
            out_specs=pl.BlockSpec((B, I), lambda i, be, ac: (i, 0)),
        ),
        out_shape=jax.ShapeDtypeStruct((Mtot, I), jnp.bfloat16),
        compiler_params=pltpu.CompilerParams(
            dimension_semantics=("arbitrary",)),
    )(bexp, acts, xs, W1g, W1l, b1g, b1l)

    ysw = pl.pallas_call(
        _mlp2_kernel,
        grid_spec=pltpu.PrefetchScalarGridSpec(
            num_scalar_prefetch=1,
            grid=(NB,),
            in_specs=[
                pl.BlockSpec((B, I), lambda i, be: (i, 0)),
                pl.BlockSpec((1, I, H), lambda i, be: (be[i], 0, 0)),
                pl.BlockSpec((1, 1, H), lambda i, be: (be[i], 0, 0)),
                pl.BlockSpec((B, 8), lambda i, be: (i, 0)),
            ],
            out_specs=pl.BlockSpec((B, H), lambda i, be: (i, 0)),
        ),
        out_shape=jax.ShapeDtypeStruct((Mtot, H), jnp.float32),
        compiler_params=pltpu.CompilerParams(
            dimension_semantics=("arbitrary",)),
    )(bexp, h, W2, b2, ws8)
    return ysw


_SC_MESH = plsc.VectorSubcoreMesh(core_axis_name="c", subcore_axis_name="s")
_NW = 32


def _sc_gather(x, perm, Mtot):
    T, H = x.shape
    per_w = Mtot // _NW
    CH = 32
    n_ch = per_w // CH

    @functools.partial(
        pl.kernel, mesh=_SC_MESH,
        out_type=jax.ShapeDtypeStruct((Mtot, H), jnp.float32),
        scratch_types=[
            pltpu.VMEM((CH,), jnp.int32),
            pltpu.VMEM((CH, H), jnp.float32),
            pltpu.SemaphoreType.DMA,
        ],
    )
    def k(x_hbm, perm_hbm, xs_hbm, idx_v, rows_v, sem):
        wid = lax.axis_index("s") * 2 + lax.axis_index("c")
        base = wid * per_w

        def body(c, carry):
            r0 = base + c * CH
            pltpu.sync_copy(perm_hbm.at[pl.ds(r0, CH)], idx_v)
            pltpu.async_copy(x_hbm.at[idx_v], rows_v, sem).wait()
            pltpu.sync_copy(rows_v, xs_hbm.at[pl.ds(r0, CH)])
            return carry

        lax.fori_loop(0, n_ch, body, 0)

    return k(x, perm)


def _sc_combine(ysw, pos0, pos1, T, Mexp):
    Mtot, H = ysw.shape
    per_w = T // _NW
    CH = 16
    n_ch = per_w // CH
    NV = (CH * H) // 16

    @functools.partial(
        pl.kernel, mesh=_SC_MESH,
        out_type=jax.ShapeDtypeStruct((T, H), jnp.float32),
        scratch_types=[
            pltpu.VMEM((CH,), jnp.int32),
            pltpu.VMEM((CH,), jnp.int32),
            pltpu.VMEM((CH, H), jnp.float32),
            pltpu.VMEM((CH, H), jnp.float32),
            pltpu.VMEM((CH, H), jnp.float32),
            pltpu.SemaphoreType.DMA,
            pltpu.SemaphoreType.DMA,
            pltpu.SemaphoreType.DMA,
        ],
    )
    def k(ysw_hbm, p0_hbm, p1_hbm, out_hbm, i0_v, i1_v, a_v, b_v, c_v,
          s0, s1, s2):
        wid = lax.axis_index("s") * 2 + lax.axis_index("c")
        base = wid * per_w

        def body(j, carry):
            t0 = base + j * CH
            pltpu.sync_copy(p0_hbm.at[pl.ds(t0, CH)], i0_v)
            pltpu.sync_copy(p1_hbm.at[pl.ds(t0, CH)], i1_v)
            cp0 = pltpu.async_copy(ysw_hbm.at[i0_v], a_v, s0)
            cp1 = pltpu.async_copy(ysw_hbm.at[i1_v], b_v, s1)
            cp2 = pltpu.async_copy(
                ysw_hbm.at[pl.ds(Mexp + t0, CH)], c_v, s2)
            cp0.wait()
            cp1.wait()
            cp2.wait()

            def add_row(r, carry2):
                def add_piece(i, carry3):
                    sl = pl.ds(i * 16, 16)
                    c_v[r, sl] = a_v[r, sl] + b_v[r, sl] + c_v[r, sl]
                    return carry3
                return lax.fori_loop(0, H // 16, add_piece, carry2)

            lax.fori_loop(0, CH, add_row, 0)
            pltpu.sync_copy(c_v, out_hbm.at[pl.ds(t0, CH)])
            return carry

        lax.fori_loop(0, n_ch, body, 0)

    return k(ysw, pos0, pos1)


def kernel(x, Wg, bg, sW1, sb1, sW2, sb2, s_alpha, s_gate_scale, s_up_shift,
           s_gc_raw, s_uc_raw, eW1, eb1, eW2, eb2, e_alpha, e_gate_scale,
           e_up_shift, e_gc_raw, e_uc_raw):
    T, H = x.shape
    I = sW2.shape[0]
    Mexp = K * T + E * B
    Mtot = Mexp + T
    NB = Mtot // B

    W1s = jnp.concatenate([eW1, sW1[None]], axis=0)
    W1g = W1s[:, :, 0::2].astype(jnp.bfloat16)
    W1l = W1s[:, :, 1::2].astype(jnp.bfloat16)
    b1s = jnp.concatenate([eb1, sb1[None]], axis=0)
    b1g = b1s[:, None, 0::2]
    b1l = b1s[:, None, 1::2]
    W2s = jnp.concatenate([eW2, sW2[None]], axis=0).astype(jnp.bfloat16)
    b2s = jnp.concatenate([eb2, sb2[None]], axis=0)[:, None, :]
    acts = jnp.concatenate([
        jnp.concatenate([e_alpha, e_gate_scale, e_up_shift, e_gc_raw,
                         e_uc_raw], axis=1),
        jnp.stack([s_alpha, s_gate_scale, s_up_shift, s_gc_raw,
                   s_uc_raw], axis=1),
    ], axis=0)

    route = _route(x, Wg, bg)

    i1 = route[:, 0].astype(jnp.int32)
    i2 = route[:, 1].astype(jnp.int32)
    eid = jnp.stack([i1, i2], 1).reshape(-1)
    wts = jnp.stack([route[:, 2], route[:, 3]], 1).reshape(-1)
    oh = (eid[:, None] == jnp.arange(E)[None, :]).astype(jnp.int32)
    cum = jnp.cumsum(oh, axis=0)
    rank = ((cum - oh) * oh).sum(1)
    g = cum[-1]
    gp = ((g + B - 1) // B) * B
    base = jnp.concatenate([jnp.zeros((1,), jnp.int32),
                            jnp.cumsum(gp)])[:E]
    p = base[eid] + rank
    tok = jnp.arange(2 * T, dtype=jnp.int32) // 2
    perm = jnp.zeros((Mtot,), jnp.int32).at[p].set(tok)
    perm = perm.at[Mexp:].set(jnp.arange(T, dtype=jnp.int32))
    wsort = jnp.zeros((Mtot,), jnp.float32).at[p].set(wts)
    wsort = wsort.at[Mexp:].set(1.0)
    bid = jnp.arange(NB, dtype=jnp.int32)
    bexp = jnp.full((NB,), E, jnp.int32)
    bb = base // B
    gpb = gp // B
    for e in range(E):
        bexp = jnp.where((bid >= bb[e]) & (bid < bb[e] + gpb[e]), e, bexp)

    xs = _sc_gather(x, perm, Mtot)
    ws8 = jnp.broadcast_to(wsort[:, None], (Mtot, 8))

    ysw = _grouped_mlp(xs, bexp, acts, W1g, W1l, b1g, b1l, W2s, b2s, ws8)

    pos0 = p[0::2]
    pos1 = p[1::2]
    return _sc_combine(ysw, pos0, pos1, T, Mexp)


# speedup vs baseline: 3.6781x; 3.6781x over previous
import functools

import jax
import jax.numpy as jnp
from jax import lax
from jax.experimental import pallas as pl
from jax.experimental.pallas import tpu as pltpu
from jax.experimental.pallas import tpu_sc as plsc

E = 8
K = 2
NEXP = E + 1
B = 128
BT = 256


def _router_kernel(x_ref, wg_ref, bg_ref, out_ref):
    logits = jnp.dot(x_ref[...], wg_ref[...],
                     preferred_element_type=jnp.float32) + bg_ref[0, :]
    lane = jax.lax.broadcasted_iota(jnp.int32, logits.shape, 1)
    big = jnp.int32(10**9)
    m1 = jnp.max(logits, axis=1, keepdims=True)
    i1 = jnp.min(jnp.where(logits >= m1, lane, big), axis=1, keepdims=True)
    l2 = jnp.where(lane == i1, -jnp.inf, logits)
    m2 = jnp.max(l2, axis=1, keepdims=True)
    i2 = jnp.min(jnp.where(l2 >= m2, lane, big), axis=1, keepdims=True)
    w1 = jax.nn.sigmoid(m1 - m2)
    w2 = jax.nn.sigmoid(m2 - m1)
    out = (jnp.where(lane == 0, i1.astype(jnp.float32), 0.0)
           + jnp.where(lane == 1, i2.astype(jnp.float32), 0.0)
           + jnp.where(lane == 2, w1, 0.0)
           + jnp.where(lane == 3, w2, 0.0))
    out_ref[...] = out[:, :8]


def _route(x, Wg, bg):
    T, H = x.shape
    Wgp = jnp.zeros((H, 128), jnp.float32).at[:, :E].set(Wg)
    bgp = jnp.full((1, 128), -1e30, jnp.float32).at[0, :E].set(bg)
    return pl.pallas_call(
        _router_kernel,
        grid=(T // BT,),
        in_specs=[
            pl.BlockSpec((BT, H), lambda i: (i, 0)),
            pl.BlockSpec((H, 128), lambda i: (0, 0)),
            pl.BlockSpec((1, 128), lambda i: (0, 0)),
        ],
        out_specs=pl.BlockSpec((BT, 8), lambda i: (i, 0)),
        out_shape=jax.ShapeDtypeStruct((T, 8), jnp.float32),
    )(x, Wgp, bgp)


def _mlp1_kernel(bexp_ref, acts_ref, xs_ref, w1_ref, b1_ref, h_ref):
    e = bexp_ref[pl.program_id(0)]
    xb = xs_ref[...].astype(jnp.bfloat16)
    t = jnp.dot(xb, w1_ref[0], preferred_element_type=jnp.float32)
    t = t + b1_ref[0, 0, :]
    tl = pltpu.roll(t, t.shape[1] - 1, 1)
    alpha = acts_ref[e, 0]
    gsc = acts_ref[e, 1]
    ush = acts_ref[e, 2]
    gc = jnp.log1p(jnp.exp(jnp.full(t.shape, acts_ref[e, 3], jnp.float32)))
    uc = jnp.log1p(jnp.exp(jnp.full(t.shape, acts_ref[e, 4], jnp.float32)))
    xg = jnp.clip(t, -gc, gc)
    xl = jnp.clip(tl, -uc, uc)
    og = xg * jax.nn.sigmoid(xg * alpha) * gsc
    hfull = og * (xl + ush)
    lane = jax.lax.broadcasted_iota(jnp.int32, t.shape, 1)
    h_ref[...] = jnp.where(lane % 2 == 0, hfull, 0.0).astype(jnp.bfloat16)


def _mlp2_kernel(bexp_ref, h_ref, w2_ref, b2_ref, ws_ref, y_ref):
    y = jnp.dot(h_ref[...], w2_ref[0], preferred_element_type=jnp.float32)
    y = y + b2_ref[0, 0, :]
    y_ref[...] = y * ws_ref[...][:, :1]


def _grouped_mlp(xs, bexp, acts, W1, b1, W2x, b2, ws8):
    Mtot, H = xs.shape
    I2 = W1.shape[2]
    NB = Mtot // B
    h = pl.pallas_call(
        _mlp1_kernel,
        grid_spec=pltpu.PrefetchScalarGridSpec(
            num_scalar_prefetch=2,
            grid=(NB,),
            in_specs=[
                pl.BlockSpec((B, H), lambda i, be, ac: (i, 0)),
                pl.BlockSpec((1, H, I2), lambda i, be, ac: (be[i], 0, 0)),
                pl.BlockSpec((1, 1, I2), lambda i, be, ac: (be[i], 0, 0)),
            ],
            out_specs=pl.BlockSpec((B, I2), lambda i, be, ac: (i, 0)),
        ),
        out_shape=jax.ShapeDtypeStruct((Mtot, I2), jnp.bfloat16),
        compiler_params=pltpu.CompilerParams(
            dimension_semantics=("arbitrary",)),
    )(bexp, acts, xs, W1, b1)

    ysw = pl.pallas_call(
        _mlp2_kernel,
        grid_spec=pltpu.PrefetchScalarGridSpec(
            num_scalar_prefetch=1,
            grid=(NB,),
            in_specs=[
                pl.BlockSpec((B, I2), lambda i, be: (i, 0)),
                pl.BlockSpec((1, I2, H), lambda i, be: (be[i], 0, 0)),
                pl.BlockSpec((1, 1, H), lambda i, be: (be[i], 0, 0)),
                pl.BlockSpec((B, 8), lambda i, be: (i, 0)),
            ],
            out_specs=pl.BlockSpec((B, H), lambda i, be: (i, 0)),
        ),
        out_shape=jax.ShapeDtypeStruct((Mtot, H), jnp.float32),
        compiler_params=pltpu.CompilerParams(
            dimension_semantics=("arbitrary",)),
    )(bexp, h, W2x, b2, ws8)
    return ysw


_SC_MESH = plsc.VectorSubcoreMesh(core_axis_name="c", subcore_axis_name="s")
_NW = 32


def _sc_gather(x, perm, Mtot):
    T, H = x.shape
    per_w = Mtot // _NW
    CH = 32
    n_ch = per_w // CH

    @functools.partial(
        pl.kernel, mesh=_SC_MESH,
        out_type=jax.ShapeDtypeStruct((Mtot, H), jnp.float32),
        scratch_types=[
            pltpu.VMEM((CH,), jnp.int32),
            pltpu.VMEM((CH, H), jnp.float32),
            pltpu.SemaphoreType.DMA,
        ],
    )
    def k(x_hbm, perm_hbm, xs_hbm, idx_v, rows_v, sem):
        wid = lax.axis_index("s") * 2 + lax.axis_index("c")
        base = wid * per_w

        def body(c, carry):
            r0 = base + c * CH
            pltpu.sync_copy(perm_hbm.at[pl.ds(r0, CH)], idx_v)
            pltpu.async_copy(x_hbm.at[idx_v], rows_v, sem).wait()
            pltpu.sync_copy(rows_v, xs_hbm.at[pl.ds(r0, CH)])
            return carry

        lax.fori_loop(0, n_ch, body, 0)

    return k(x, perm)


def _sc_combine(ysw, pos0, pos1, T, Mexp):
    Mtot, H = ysw.shape
    per_w = T // _NW
    CH = 16
    n_ch = per_w // CH
    NV = (CH * H) // 16

    @functools.partial(
        pl.kernel, mesh=_SC_MESH,
        out_type=jax.ShapeDtypeStruct((T, H), jnp.float32),
        scratch_types=[
            pltpu.VMEM((CH,), jnp.int32),
            pltpu.VMEM((CH,), jnp.int32),
            pltpu.VMEM((CH, H), jnp.float32),
            pltpu.VMEM((CH, H), jnp.float32),
            pltpu.VMEM((CH, H), jnp.float32),
            pltpu.SemaphoreType.DMA,
            pltpu.SemaphoreType.DMA,
            pltpu.SemaphoreType.DMA,
        ],
    )
    def k(ysw_hbm, p0_hbm, p1_hbm, out_hbm, i0_v, i1_v, a_v, b_v, c_v,
          s0, s1, s2):
        wid = lax.axis_index("s") * 2 + lax.axis_index("c")
        base = wid * per_w

        def body(j, carry):
            t0 = base + j * CH
            pltpu.sync_copy(p0_hbm.at[pl.ds(t0, CH)], i0_v)
            pltpu.sync_copy(p1_hbm.at[pl.ds(t0, CH)], i1_v)
            cp0 = pltpu.async_copy(ysw_hbm.at[i0_v], a_v, s0)
            cp1 = pltpu.async_copy(ysw_hbm.at[i1_v], b_v, s1)
            cp2 = pltpu.async_copy(
                ysw_hbm.at[pl.ds(Mexp + t0, CH)], c_v, s2)
            cp0.wait()
            cp1.wait()
            cp2.wait()

            def add_row(r, carry2):
                def add_piece(i, carry3):
                    sl = pl.ds(i * 16, 16)
                    c_v[r, sl] = a_v[r, sl] + b_v[r, sl] + c_v[r, sl]
                    return carry3
                return lax.fori_loop(0, H // 16, add_piece, carry2)

            lax.fori_loop(0, CH, add_row, 0)
            pltpu.sync_copy(c_v, out_hbm.at[pl.ds(t0, CH)])
            return carry

        lax.fori_loop(0, n_ch, body, 0)

    return k(ysw, pos0, pos1)


def kernel(x, Wg, bg, sW1, sb1, sW2, sb2, s_alpha, s_gate_scale, s_up_shift,
           s_gc_raw, s_uc_raw, eW1, eb1, eW2, eb2, e_alpha, e_gate_scale,
           e_up_shift, e_gc_raw, e_uc_raw):
    T, H = x.shape
    I = sW2.shape[0]
    Mexp = K * T + E * B
    Mtot = Mexp + T
    NB = Mtot // B

    W1s = jnp.concatenate([eW1, sW1[None]], axis=0)
    W1b = W1s.astype(jnp.bfloat16)
    b1s = jnp.concatenate([eb1, sb1[None]], axis=0)[:, None, :]
    W2s = jnp.concatenate([eW2, sW2[None]], axis=0).astype(jnp.bfloat16)
    W2x = jnp.stack([W2s, jnp.zeros_like(W2s)], axis=2)
    W2x = W2x.reshape(NEXP, 2 * I, H)
    b2s = jnp.concatenate([eb2, sb2[None]], axis=0)[:, None, :]
    acts = jnp.concatenate([
        jnp.concatenate([e_alpha, e_gate_scale, e_up_shift, e_gc_raw,
                         e_uc_raw], axis=1),
        jnp.stack([s_alpha, s_gate_scale, s_up_shift, s_gc_raw,
                   s_uc_raw], axis=1),
    ], axis=0)

    route = _route(x, Wg, bg)

    i1 = route[:, 0].astype(jnp.int32)
    i2 = route[:, 1].astype(jnp.int32)
    eid = jnp.stack([i1, i2], 1).reshape(-1)
    wts = jnp.stack([route[:, 2], route[:, 3]], 1).reshape(-1)
    oh = (eid[:, None] == jnp.arange(E)[None, :]).astype(jnp.int32)
    cum = jnp.cumsum(oh, axis=0)
    rank = ((cum - oh) * oh).sum(1)
    g = cum[-1]
    gp = ((g + B - 1) // B) * B
    base = jnp.concatenate([jnp.zeros((1,), jnp.int32),
                            jnp.cumsum(gp)])[:E]
    p = base[eid] + rank
    tok = jnp.arange(2 * T, dtype=jnp.int32) // 2
    perm = jnp.zeros((Mtot,), jnp.int32).at[p].set(tok)
    perm = perm.at[Mexp:].set(jnp.arange(T, dtype=jnp.int32))
    wsort = jnp.zeros((Mtot,), jnp.float32).at[p].set(wts)
    wsort = wsort.at[Mexp:].set(1.0)
    bid = jnp.arange(NB, dtype=jnp.int32)
    bexp = jnp.full((NB,), E, jnp.int32)
    bb = base // B
    gpb = gp // B
    for e in range(E):
        bexp = jnp.where((bid >= bb[e]) & (bid < bb[e] + gpb[e]), e, bexp)

    xs = _sc_gather(x, perm, Mtot)
    ws8 = jnp.broadcast_to(wsort[:, None], (Mtot, 8))

    ysw = _grouped_mlp(xs, bexp, acts, W1b, b1s, W2x, b2s, ws8)

    pos0 = p[0::2]
    pos1 = p[1::2]
    return _sc_combine(ysw, pos0, pos1, T, Mexp)
